# Initial kernel scaffold; baseline (speedup 1.0000x reference)
import jax
import jax.numpy as jnp
from jax.experimental import pallas as pl


def kernel(lig_feat, prot_feat, mask, W_lig, b_lig, W_prot, b_prot, Wq, bq, Wk, bk, Wv, bv, Wo, bo, W1, b1, W2, b2, ln1_g, ln1_b, ln2_g, ln2_b, lign_g, lign_b, protn_g, protn_b, Wr, br, We, be):
    raise NotImplementedError("write your pallas kernel here")


# speedup vs baseline: 1.5411x; 1.5411x over previous
import functools
import math

import jax
import jax.numpy as jnp
from jax.experimental import pallas as pl
from jax.experimental.pallas import tpu as pltpu

B, L, LIG_D, PROT_D, H, NH, NL, E, TOPK = 4, 2048, 384, 1536, 256, 8, 2, 64, 2
DH = H // NH
S = L + 1
P = 2112
NCHUNK = 4
CHUNK = P // NCHUNK
EPS = 1e-5
SCALE = 1.0 / math.sqrt(DH)


def _bf(x):
    return x.astype(jnp.bfloat16)


def _dot(a, b):
    return jax.lax.dot_general(
        _bf(a), _bf(b), (((1,), (0,)), ((), ())),
        preferred_element_type=jnp.float32)


def _ln(x, g, b):
    mu = jnp.mean(x, axis=-1, keepdims=True)
    xc = x - mu
    v = jnp.mean(xc * xc, axis=-1, keepdims=True)
    return xc * jax.lax.rsqrt(v + EPS) * g + b


def _fwd_kernel(lig_ref, prot_ref, mask_ref, cos_ref, sin_ref,
                wlig_ref, blig_ref, wprot_ref, bprot_ref,
                wq_ref, bq_ref, wk_ref, bk_ref, wv_ref, bv_ref,
                wo_ref, bo_ref, w1_ref, b1_ref, w2_ref, b2_ref,
                ln1g_ref, ln1b_ref, ln2g_ref, ln2b_ref,
                ligng_ref, lignb_ref, protng_ref, protnb_ref,
                wr_ref, br_ref, wet_ref, be_ref, out_ref):
    lig_feat = lig_ref[0]
    lig = _dot(lig_feat, wlig_ref[...]) + blig_ref[...]
    cos = cos_ref[...]
    sin = sin_ref[...]
    x1 = lig[:, :H // 2]
    x2 = lig[:, H // 2:]
    lig = jnp.concatenate([x1 * cos - x2 * sin, x1 * sin + x2 * cos], axis=1)
    prot = _dot(prot_ref[0], wprot_ref[...]) + bprot_ref[...]
    pad = jnp.zeros((P - S, H), jnp.float32)
    x = jnp.concatenate([prot, lig, pad], axis=0)

    col = jax.lax.broadcasted_iota(jnp.int32, (1, P), 1)
    key_bias = jnp.where(col < S, 0.0, -1e9).astype(jnp.float32)
    lane = jax.lax.broadcasted_iota(jnp.int32, (1, H), 1)

    for i in range(NL):
        q = _dot(x, wq_ref[i]) + bq_ref[i]
        k = _dot(x, wk_ref[i]) + bk_ref[i]
        v = _dot(x, wv_ref[i]) + bv_ref[i]
        kt = jnp.transpose(_bf(k))
        v_bf = _bf(v)
        q_bf = _bf(q)

        o_chunks = []
        for c in range(NCHUNK):
            q_c = q_bf[c * CHUNK:(c + 1) * CHUNK, :]

            def head_body(h, o_c):
                mh = (lane // DH == h).astype(jnp.bfloat16)
                s = jax.lax.dot_general(
                    q_c * mh, kt, (((1,), (0,)), ((), ())),
                    preferred_element_type=jnp.float32)
                s = s * SCALE + key_bias
                m = jnp.max(s, axis=-1, keepdims=True)
                p = jnp.exp(s - m)
                a = p / jnp.sum(p, axis=-1, keepdims=True)
                o_h = jax.lax.dot_general(
                    _bf(a), v_bf * mh, (((1,), (0,)), ((), ())),
                    preferred_element_type=jnp.float32)
                return o_c + o_h

            o_c = jax.lax.fori_loop(
                0, NH, head_body, jnp.zeros((CHUNK, H), jnp.float32))
            o_chunks.append(o_c)
        o = jnp.concatenate(o_chunks, axis=0)

        h_att = _dot(o, wo_ref[i]) + bo_ref[i]
        x = _ln(x + h_att, ln1g_ref[i], ln1b_ref[i])
        u = _dot(x, w1_ref[i]) + b1_ref[i]
        u = jax.nn.gelu(u)
        h_ff = _dot(u, w2_ref[i]) + b2_ref[i]
        x = _ln(x + h_ff, ln2g_ref[i], ln2b_ref[i])
        if i < NL - 1:
            row = jax.lax.broadcasted_iota(jnp.int32, (P, 1), 0)
            g = jnp.where(row == 0, protng_ref[i], ligng_ref[i])
            b = jnp.where(row == 0, protnb_ref[i], lignb_ref[i])
            x = _ln(x, g, b)

    tok = x[1:S, :]
    logits = _dot(tok, wr_ref[...]) + br_ref[...]
    eout = _dot(tok, wet_ref[...]) + be_ref[...]

    eidx = jax.lax.broadcasted_iota(jnp.int32, (L, E), 1)
    m1 = jnp.max(logits, axis=-1, keepdims=True)
    i1 = jnp.min(jnp.where(logits == m1, eidx, E), axis=-1, keepdims=True)
    oh1 = (eidx == i1)
    rest = jnp.where(oh1, -jnp.inf, logits)
    m2 = jnp.max(rest, axis=-1, keepdims=True)
    i2 = jnp.min(jnp.where(rest == m2, eidx, E), axis=-1, keepdims=True)
    oh2 = (eidx == i2)
    w2nd = jnp.exp(m2 - m1)
    denom = 1.0 + w2nd
    eo1 = jnp.sum(jnp.where(oh1, eout, 0.0), axis=-1, keepdims=True)
    eo2 = jnp.sum(jnp.where(oh2, eout, 0.0), axis=-1, keepdims=True)
    out_tok = (eo1 + w2nd * eo2) / denom

    mrow = mask_ref[0]
    mcol = jnp.transpose(mrow)
    total = jnp.sum(out_tok * mcol)
    cnt = jnp.maximum(jnp.sum(mrow), 1.0)
    out_ref[0] = jnp.full((1, 128), total / cnt, jnp.float32)


@jax.jit
def kernel(lig_feat, prot_feat, mask, W_lig, b_lig, W_prot, b_prot,
           Wq, bq, Wk, bk, Wv, bv, Wo, bo, W1, b1, W2, b2,
           ln1_g, ln1_b, ln2_g, ln2_b, lign_g, lign_b, protn_g, protn_b,
           Wr, br, We, be):
    half = H // 2
    inv_freq = 1.0 / (10000.0 ** (jnp.arange(half, dtype=jnp.float32) / half))
    ang = jnp.arange(L, dtype=jnp.float32)[:, None] * inv_freq[None, :]
    cos_t, sin_t = jnp.cos(ang), jnp.sin(ang)

    r2 = lambda a: a.reshape(NL, 1, H)
    args = (
        lig_feat, prot_feat.reshape(B, 1, PROT_D),
        mask.astype(jnp.float32).reshape(B, 1, L),
        cos_t, sin_t,
        W_lig, b_lig.reshape(1, H), W_prot, b_prot.reshape(1, H),
        Wq, r2(bq), Wk, r2(bk), Wv, r2(bv), Wo, r2(bo),
        W1, b1.reshape(NL, 1, 4 * H), W2, r2(b2),
        r2(ln1_g), r2(ln1_b), r2(ln2_g), r2(ln2_b),
        lign_g.reshape(NL - 1, 1, H), lign_b.reshape(NL - 1, 1, H),
        protn_g.reshape(NL - 1, 1, H), protn_b.reshape(NL - 1, 1, H),
        Wr, br.reshape(1, E), We.T, be.reshape(1, E),
    )

    def bspec(shape, batched):
        if batched:
            return pl.BlockSpec((1,) + shape[1:],
                                lambda i: (i,) + (0,) * (len(shape) - 1))
        return pl.BlockSpec(shape, lambda i: (0,) * len(shape))

    in_specs = [
        bspec(args[0].shape, True),
        bspec(args[1].shape, True),
        bspec(args[2].shape, True),
    ] + [bspec(a.shape, False) for a in args[3:]]

    out = pl.pallas_call(
        _fwd_kernel,
        grid=(B,),
        in_specs=in_specs,
        out_specs=pl.BlockSpec((1, 1, 128), lambda i: (i, 0, 0)),
        out_shape=jax.ShapeDtypeStruct((B, 1, 128), jnp.float32),
        compiler_params=pltpu.CompilerParams(
            dimension_semantics=("parallel",)),
    )(*args)
    return out[:, 0, 0]


# speedup vs baseline: 1.7013x; 1.1040x over previous
import functools
import math

import jax
import jax.numpy as jnp
from jax.experimental import pallas as pl
from jax.experimental.pallas import tpu as pltpu

B, L, LIG_D, PROT_D, H, NH, NL, E, TOPK = 4, 2048, 384, 1536, 256, 8, 2, 64, 2
DH = H // NH
S = L + 1
P = 2112
NCHUNK = 4
CHUNK = P // NCHUNK
EPS = 1e-5
SCALE = 1.0 / math.sqrt(DH)


def _bf(x):
    return x.astype(jnp.bfloat16)


def _dot(a, b):
    return jax.lax.dot_general(
        _bf(a), _bf(b), (((1,), (0,)), ((), ())),
        preferred_element_type=jnp.float32)


def _ln(x, g, b):
    mu = jnp.mean(x, axis=-1, keepdims=True)
    xc = x - mu
    v = jnp.mean(xc * xc, axis=-1, keepdims=True)
    return xc * jax.lax.rsqrt(v + EPS) * g + b


def _fwd_kernel(lig_ref, prot_ref, mask_ref, cos_ref, sin_ref,
                wlig_ref, blig_ref, wprot_ref, bprot_ref,
                wq_ref, bq_ref, wk_ref, bk_ref, wv_ref, bv_ref,
                wo_ref, bo_ref, w1_ref, b1_ref, w2_ref, b2_ref,
                ln1g_ref, ln1b_ref, ln2g_ref, ln2b_ref,
                ligng_ref, lignb_ref, protng_ref, protnb_ref,
                wr_ref, br_ref, wet_ref, be_ref, out_ref):
    lig_feat = lig_ref[0]
    lig = _dot(lig_feat, wlig_ref[...]) + blig_ref[...]
    cos = cos_ref[...]
    sin = sin_ref[...]
    x1 = lig[:, :H // 2]
    x2 = lig[:, H // 2:]
    lig = jnp.concatenate([x1 * cos - x2 * sin, x1 * sin + x2 * cos], axis=1)
    prot = _dot(prot_ref[0], wprot_ref[...]) + bprot_ref[...]
    pad = jnp.zeros((P - S, H), jnp.float32)
    x = jnp.concatenate([prot, lig, pad], axis=0)

    col = jax.lax.broadcasted_iota(jnp.int32, (1, P), 1)
    key_bias = jnp.where(col < S, 0.0, -1e9).astype(jnp.float32)
    lane = jax.lax.broadcasted_iota(jnp.int32, (1, H), 1)

    for i in range(NL):
        q = _dot(x, wq_ref[i]) + bq_ref[i]
        k = _dot(x, wk_ref[i]) + bk_ref[i]
        v = _dot(x, wv_ref[i]) + bv_ref[i]
        kt = jnp.transpose(_bf(k))
        v_bf = _bf(v)
        q_bf = _bf(q * SCALE)

        o_chunks = []
        for c in range(NCHUNK):
            q_c = q_bf[c * CHUNK:(c + 1) * CHUNK, :]

            def head_body(h, o_c):
                mh = (lane // DH == h).astype(jnp.bfloat16)
                s = jax.lax.dot_general(
                    q_c * mh, kt, (((1,), (0,)), ((), ())),
                    preferred_element_type=jnp.float32)
                s = s + key_bias
                m = jnp.max(s, axis=-1, keepdims=True)
                p = jnp.exp(s - m)
                r = 1.0 / jnp.sum(p, axis=-1, keepdims=True)
                o_h = jax.lax.dot_general(
                    _bf(p), v_bf * mh, (((1,), (0,)), ((), ())),
                    preferred_element_type=jnp.float32)
                return o_c + o_h * r

            o_c = jax.lax.fori_loop(
                0, NH, head_body, jnp.zeros((CHUNK, H), jnp.float32))
            o_chunks.append(o_c)
        o = jnp.concatenate(o_chunks, axis=0)

        h_att = _dot(o, wo_ref[i]) + bo_ref[i]
        x = _ln(x + h_att, ln1g_ref[i], ln1b_ref[i])
        u = _dot(x, w1_ref[i]) + b1_ref[i]
        u = jax.nn.gelu(u)
        h_ff = _dot(u, w2_ref[i]) + b2_ref[i]
        x = _ln(x + h_ff, ln2g_ref[i], ln2b_ref[i])
        if i < NL - 1:
            row = jax.lax.broadcasted_iota(jnp.int32, (P, 1), 0)
            g = jnp.where(row == 0, protng_ref[i], ligng_ref[i])
            b = jnp.where(row == 0, protnb_ref[i], lignb_ref[i])
            x = _ln(x, g, b)

    tok = x[1:S, :]
    logits = _dot(tok, wr_ref[...]) + br_ref[...]
    eout = _dot(tok, wet_ref[...]) + be_ref[...]

    eidx = jax.lax.broadcasted_iota(jnp.int32, (L, E), 1)
    m1 = jnp.max(logits, axis=-1, keepdims=True)
    i1 = jnp.min(jnp.where(logits == m1, eidx, E), axis=-1, keepdims=True)
    oh1 = (eidx == i1)
    rest = jnp.where(oh1, -jnp.inf, logits)
    m2 = jnp.max(rest, axis=-1, keepdims=True)
    i2 = jnp.min(jnp.where(rest == m2, eidx, E), axis=-1, keepdims=True)
    oh2 = (eidx == i2)
    w2nd = jnp.exp(m2 - m1)
    denom = 1.0 + w2nd
    eo1 = jnp.sum(jnp.where(oh1, eout, 0.0), axis=-1, keepdims=True)
    eo2 = jnp.sum(jnp.where(oh2, eout, 0.0), axis=-1, keepdims=True)
    out_tok = (eo1 + w2nd * eo2) / denom

    mrow = mask_ref[0]
    mcol = jnp.transpose(mrow)
    total = jnp.sum(out_tok * mcol)
    cnt = jnp.maximum(jnp.sum(mrow), 1.0)
    out_ref[0] = jnp.full((1, 128), total / cnt, jnp.float32)


@jax.jit
def kernel(lig_feat, prot_feat, mask, W_lig, b_lig, W_prot, b_prot,
           Wq, bq, Wk, bk, Wv, bv, Wo, bo, W1, b1, W2, b2,
           ln1_g, ln1_b, ln2_g, ln2_b, lign_g, lign_b, protn_g, protn_b,
           Wr, br, We, be):
    half = H // 2
    inv_freq = 1.0 / (10000.0 ** (jnp.arange(half, dtype=jnp.float32) / half))
    ang = jnp.arange(L, dtype=jnp.float32)[:, None] * inv_freq[None, :]
    cos_t, sin_t = jnp.cos(ang), jnp.sin(ang)

    r2 = lambda a: a.reshape(NL, 1, H)
    args = (
        lig_feat, prot_feat.reshape(B, 1, PROT_D),
        mask.astype(jnp.float32).reshape(B, 1, L),
        cos_t, sin_t,
        W_lig, b_lig.reshape(1, H), W_prot, b_prot.reshape(1, H),
        Wq, r2(bq), Wk, r2(bk), Wv, r2(bv), Wo, r2(bo),
        W1, b1.reshape(NL, 1, 4 * H), W2, r2(b2),
        r2(ln1_g), r2(ln1_b), r2(ln2_g), r2(ln2_b),
        lign_g.reshape(NL - 1, 1, H), lign_b.reshape(NL - 1, 1, H),
        protn_g.reshape(NL - 1, 1, H), protn_b.reshape(NL - 1, 1, H),
        Wr, br.reshape(1, E), We.T, be.reshape(1, E),
    )

    def bspec(shape, batched):
        if batched:
            return pl.BlockSpec((1,) + shape[1:],
                                lambda i: (i,) + (0,) * (len(shape) - 1))
        return pl.BlockSpec(shape, lambda i: (0,) * len(shape))

    in_specs = [
        bspec(args[0].shape, True),
        bspec(args[1].shape, True),
        bspec(args[2].shape, True),
    ] + [bspec(a.shape, False) for a in args[3:]]

    out = pl.pallas_call(
        _fwd_kernel,
        grid=(B,),
        in_specs=in_specs,
        out_specs=pl.BlockSpec((1, 1, 128), lambda i: (i, 0, 0)),
        out_shape=jax.ShapeDtypeStruct((B, 1, 128), jnp.float32),
        compiler_params=pltpu.CompilerParams(
            dimension_semantics=("parallel",)),
    )(*args)
    return out[:, 0, 0]
